# baseline (speedup 1.0000x reference)
import jax
import jax.numpy as jnp
from jax import lax
from jax.experimental import pallas as pl
from jax.experimental.pallas import tpu as pltpu
from jax.experimental.pallas import tpu_sc as plsc

D = 64
BATCH = 4096
SEQ = 200
HALF0 = 104
HALF1 = 96
NB = 4
AUXZ = 64
AUXP = 8

_info = plsc.get_sparse_core_info()
NC, NS, L = _info.num_cores, _info.num_subcores, _info.num_lanes
NW = NC * NS
SEQS_PER_W = BATCH // NW


def _body(x_hbm, p_hbm, table_hbm, aux_hbm, pos_hbm, out_hbm,
          idx_all, p_all,
          rows0, rows1, rows2, rows3,
          psem0, psem1, psem2, psem3,
          gsem0, gsem1, gsem2, gsem3,
          ssem0, ssem1, ssem2, ssem3):
    wid = lax.axis_index("s") * NC + lax.axis_index("c")
    wbase = wid * SEQS_PER_W * SEQ
    pltpu.sync_copy(x_hbm.at[pl.ds(wbase, SEQS_PER_W * SEQ)], idx_all)
    pltpu.sync_copy(p_hbm.at[pl.ds(wbase, SEQS_PER_W * SEQ)], p_all)

    rows = (rows0, rows1, rows2, rows3)
    psem = (psem0, psem1, psem2, psem3)
    gsem = (gsem0, gsem1, gsem2, gsem3)
    ssem = (ssem0, ssem1, ssem2, ssem3)

    def issue_prefill(b):
        pltpu.async_copy(pos_hbm, rows[b], psem[b])

    def wait_prefill(b):
        pltpu.make_async_copy(pos_hbm, rows[b], psem[b]).wait()

    def issue_gather(c, b):
        off = c * SEQ
        pltpu.async_copy(table_hbm.at[idx_all.at[pl.ds(off, HALF0)]],
                         rows[b].at[pl.ds(0, HALF0), :], gsem[b], add=True)
        pltpu.async_copy(table_hbm.at[idx_all.at[pl.ds(off + HALF0, HALF1)]],
                         rows[b].at[pl.ds(HALF0, HALF1), :], gsem[b], add=True)
        pltpu.async_copy(aux_hbm.at[p_all.at[pl.ds(off, HALF0)]],
                         rows[b].at[pl.ds(0, HALF0), :], gsem[b], add=True)
        pltpu.async_copy(aux_hbm.at[p_all.at[pl.ds(off + HALF0, HALF1)]],
                         rows[b].at[pl.ds(HALF0, HALF1), :], gsem[b], add=True)

    def wait_gather(b):
        pltpu.make_async_copy(out_hbm.at[pl.ds(0, SEQ), :], rows[b],
                              gsem[b]).wait()
        pltpu.make_async_copy(out_hbm.at[pl.ds(0, SEQ), :], rows[b],
                              gsem[b]).wait()

    def issue_scatter(c, b):
        pltpu.async_copy(rows[b], out_hbm.at[pl.ds(wbase + c * SEQ, SEQ), :],
                         ssem[b])

    def wait_scatter(b):
        pltpu.make_async_copy(rows[b], out_hbm.at[pl.ds(0, SEQ), :],
                              ssem[b]).wait()

    N = SEQS_PER_W
    issue_prefill(0)
    issue_prefill(1)
    wait_prefill(0)
    issue_gather(0, 0)

    def quad_body(gi, carry):
        for b in range(NB):
            c = gi * NB + b
            s1 = (b + 1) % NB
            s2 = (b + 2) % NB

            @pl.when(c + 2 < N)
            def _():
                @pl.when(c >= 2)
                def _():
                    wait_scatter(s2)
                issue_prefill(s2)

            @pl.when(c + 1 < N)
            def _():
                wait_prefill(s1)
                issue_gather(c + 1, s1)

            wait_gather(b)
            issue_scatter(c, b)
        return carry

    lax.fori_loop(0, N // NB, quad_body, 0)
    for b in range(NB):
        wait_scatter(b)


@jax.jit
def _run(xf, p, table, aux, pos_enc):
    fn = pl.kernel(
        _body,
        mesh=plsc.VectorSubcoreMesh(core_axis_name="c", subcore_axis_name="s"),
        compiler_params=pltpu.CompilerParams(use_tc_tiling_on_sc=False),
        out_type=jax.ShapeDtypeStruct((BATCH * SEQ, D), jnp.float32),
        scratch_types=(
            [pltpu.VMEM((SEQS_PER_W * SEQ,), jnp.int32)] * 2
            + [pltpu.VMEM((SEQ, D), jnp.float32)] * 4
            + [pltpu.SemaphoreType.DMA] * 12
        ),
    )
    return fn(xf, p, table, aux, pos_enc)


def kernel(x, table, pos_enc):
    xf = x.reshape(BATCH * SEQ)
    n = jnp.arange(BATCH * SEQ, dtype=jnp.int32)
    p = jnp.where(xf == 2, AUXZ + (n & (AUXP - 1)), n & (AUXZ - 1))
    aux = jnp.concatenate(
        [jnp.zeros((AUXZ, D), jnp.float32),
         jnp.tile(-table[2], (AUXP, 1))], axis=0)
    out = _run(xf, p.astype(jnp.int32), table, aux, pos_enc)
    return out.reshape(BATCH, SEQ, D)


# speedup vs baseline: 1.3453x; 1.3453x over previous
import jax
import jax.numpy as jnp
from jax import lax
from jax.experimental import pallas as pl
from jax.experimental.pallas import tpu as pltpu
from jax.experimental.pallas import tpu_sc as plsc

D = 64
BATCH = 4096
SEQ = 200
NB = 4

_info = plsc.get_sparse_core_info()
NC, NS, L = _info.num_cores, _info.num_subcores, _info.num_lanes
NW = NC * NS
SEQS_PER_W = BATCH // NW


def _body(x_hbm, q_hbm, table_hbm, posaux_hbm, out_hbm,
          idx_all, q_all,
          rows0, rows1, rows2, rows3,
          qsem0, qsem1, qsem2, qsem3,
          gsem0, gsem1, gsem2, gsem3,
          ssem0, ssem1, ssem2, ssem3):
    wid = lax.axis_index("s") * NC + lax.axis_index("c")
    wbase = wid * SEQS_PER_W * SEQ
    pltpu.sync_copy(x_hbm.at[pl.ds(wbase, SEQS_PER_W * SEQ)], idx_all)
    pltpu.sync_copy(q_hbm.at[pl.ds(wbase, SEQS_PER_W * SEQ)], q_all)

    rows = (rows0, rows1, rows2, rows3)
    qsem = (qsem0, qsem1, qsem2, qsem3)
    gsem = (gsem0, gsem1, gsem2, gsem3)
    ssem = (ssem0, ssem1, ssem2, ssem3)

    def issue_qgather(c, b):
        pltpu.async_copy(posaux_hbm.at[q_all.at[pl.ds(c * SEQ, SEQ)]],
                         rows[b], qsem[b])

    def wait_qgather(b):
        pltpu.make_async_copy(out_hbm.at[pl.ds(0, SEQ), :], rows[b],
                              qsem[b]).wait()

    def issue_gather(c, b):
        pltpu.async_copy(table_hbm.at[idx_all.at[pl.ds(c * SEQ, SEQ)]],
                         rows[b], gsem[b], add=True)

    def wait_gather(b):
        pltpu.make_async_copy(out_hbm.at[pl.ds(0, SEQ), :], rows[b],
                              gsem[b]).wait()

    def issue_scatter(c, b):
        pltpu.async_copy(rows[b], out_hbm.at[pl.ds(wbase + c * SEQ, SEQ), :],
                         ssem[b])

    def wait_scatter(b):
        pltpu.make_async_copy(rows[b], out_hbm.at[pl.ds(0, SEQ), :],
                              ssem[b]).wait()

    N = SEQS_PER_W
    issue_qgather(0, 0)
    issue_qgather(1, 1)
    wait_qgather(0)
    issue_gather(0, 0)

    def quad_body(gi, carry):
        for b in range(NB):
            c = gi * NB + b
            s1 = (b + 1) % NB
            s2 = (b + 2) % NB

            @pl.when(c + 2 < N)
            def _():
                @pl.when(c >= 2)
                def _():
                    wait_scatter(s2)
                issue_qgather(c + 2, s2)

            @pl.when(c + 1 < N)
            def _():
                wait_qgather(s1)
                issue_gather(c + 1, s1)

            wait_gather(b)
            issue_scatter(c, b)
        return carry

    lax.fori_loop(0, N // NB, quad_body, 0)
    for b in range(NB):
        wait_scatter(b)


@jax.jit
def _run(xf, q, table, posaux):
    fn = pl.kernel(
        _body,
        mesh=plsc.VectorSubcoreMesh(core_axis_name="c", subcore_axis_name="s"),
        compiler_params=pltpu.CompilerParams(use_tc_tiling_on_sc=False),
        out_type=jax.ShapeDtypeStruct((BATCH * SEQ, D), jnp.float32),
        scratch_types=(
            [pltpu.VMEM((SEQS_PER_W * SEQ,), jnp.int32)] * 2
            + [pltpu.VMEM((SEQ, D), jnp.float32)] * 4
            + [pltpu.SemaphoreType.DMA] * 12
        ),
    )
    return fn(xf, q, table, posaux)


def kernel(x, table, pos_enc):
    xf = x.reshape(BATCH * SEQ)
    n = jnp.arange(BATCH * SEQ, dtype=jnp.int32)
    s = n % SEQ
    q = jnp.where(xf == 2, SEQ + s, s).astype(jnp.int32)
    posaux = jnp.concatenate([pos_enc, pos_enc - table[2]], axis=0)
    out = _run(xf, q, table, posaux)
    return out.reshape(BATCH, SEQ, D)
